# baseline (speedup 1.0000x reference)
import functools

import jax
import jax.numpy as jnp
from jax import lax
from jax.experimental import pallas as pl
from jax.experimental.pallas import tpu as pltpu
from jax.experimental.pallas import tpu_sc as plsc

_NS = 16
_TILE = 128
_LANES = 16


def _dval_table(W, Wl, bl):
    def body(w_ref, wl_ref, bl_ref, out_ref):
        u = jnp.dot(w_ref[...], wl_ref[...], preferred_element_type=jnp.float32)
        u = jnp.maximum(u + bl_ref[...], 0.0)
        row = lax.broadcasted_iota(jnp.int32, u.shape, 0)
        out_ref[...] = jnp.where(row == 0, 0.0, u)

    V = W.shape[0]
    return pl.pallas_call(
        body,
        out_shape=jax.ShapeDtypeStruct((V, 1), jnp.float32),
    )(W, Wl, bl)


def _sc_bow(qvalb_flat, qidx_flat, didx_flat, dval_pad, V, B, L):
    CPW = B // _NS
    NT = CPW // _TILE
    NG = _TILE // _LANES
    Vp = dval_pad.shape[0]
    GW = _LANES * L
    UNR = 5
    assert L % UNR == 0

    mesh = plsc.VectorSubcoreMesh(
        core_axis_name="c", subcore_axis_name="s",
        num_cores=1, num_subcores=_NS)

    @functools.partial(
        pl.kernel,
        out_type=(jax.ShapeDtypeStruct((B,), jnp.float32),
                  jax.ShapeDtypeStruct((V, B), jnp.float32)),
        mesh=mesh,
        compiler_params=pltpu.CompilerParams(
            needs_layout_passes=False, skip_device_barrier=True),
        scratch_types=[
            pltpu.VMEM((GW,), jnp.int32),
            pltpu.VMEM((GW,), jnp.int32),
            pltpu.VMEM((Vp,), jnp.float32),
            pltpu.VMEM((V, _TILE), jnp.float32),
            pltpu.VMEM((_TILE,), jnp.float32),
            pltpu.SemaphoreType.DMA,
            pltpu.SemaphoreType.DMA,
            pltpu.SemaphoreType.DMA,
            pltpu.SemaphoreType.DMA,
        ])
    def k(qvalb_hbm, qidx_hbm, didx_hbm, dval_hbm, rel_hbm, dbow_hbm,
          buf_a, buf_b, dval_v, acc_v, rel_v, sem_a, sem_b, sem_d, sem_o):
        wid = lax.axis_index("s")
        lane = lax.iota(jnp.int32, _LANES)
        zero16 = jnp.zeros((_LANES,), jnp.float32)
        fbase = lane * L
        bufs, sems = [buf_a, buf_b], [sem_a, sem_b]

        cp_d = pltpu.async_copy(dval_hbm, dval_v, sem_d)
        cps = [pltpu.async_copy(
            didx_hbm.at[pl.ds(wid * CPW * L, GW)], buf_a, sem_a), None]

        rows_per_it = 4
        def zrow(r, carry):
            for rr in range(rows_per_it):
                for c in range(_TILE // _LANES):
                    acc_v[r * rows_per_it + rr, pl.ds(c * _LANES, _LANES)] = zero16
            return carry
        lax.fori_loop(0, V // rows_per_it, zrow, 0)
        cp_d.wait()

        cp_out = None
        for t in range(NT):
            bs = wid * CPW + t * _TILE

            for g in range(NG):
                cps[g % 2].wait()
                if g + 1 < NG:
                    nxt = (bs + (g + 1) * _LANES) * L
                    cps[(g + 1) % 2] = pltpu.async_copy(
                        didx_hbm.at[pl.ds(nxt, GW)],
                        bufs[(g + 1) % 2], sems[(g + 1) % 2])
                cur = bufs[g % 2]
                jloc = g * _LANES + lane

                def dbody(i, carry):
                    for u in range(UNR):
                        dtok = plsc.load_gather(cur, [fbase + (i * UNR + u)])
                        dv = plsc.load_gather(dval_v, [dtok])
                        plsc.addupdate_scatter(acc_v, [dtok, jloc], dv)
                    return carry
                lax.fori_loop(0, L // UNR, dbody, 0)

            cp_out = pltpu.async_copy(
                acc_v, dbow_hbm.at[:, pl.ds(bs, _TILE)], sem_o)

            for g in range(NG):
                goff = (bs + g * _LANES) * L
                pltpu.sync_copy(qidx_hbm.at[pl.ds(goff, GW)], buf_a)
                pltpu.sync_copy(qvalb_hbm.at[pl.ds(goff, GW)], buf_b)
                jloc = g * _LANES + lane

                def qbody(i, racc):
                    for u in range(UNR):
                        qtok = plsc.load_gather(buf_a, [fbase + (i * UNR + u)])
                        qvb = plsc.load_gather(buf_b, [fbase + (i * UNR + u)])
                        dcol = plsc.load_gather(acc_v, [qtok, jloc])
                        racc = racc + plsc.bitcast(qvb, jnp.float32) * dcol
                    return racc
                racc = lax.fori_loop(0, L // UNR, qbody, zero16)
                rel_v[pl.ds(g * _LANES, _LANES)] = racc

            pltpu.sync_copy(rel_v, rel_hbm.at[pl.ds(bs, _TILE)])
            cp_out.wait()

            if t + 1 < NT:
                cps[0] = pltpu.async_copy(
                    didx_hbm.at[pl.ds((wid * CPW + (t + 1) * _TILE) * L, GW)],
                    buf_a, sem_a)
                lax.fori_loop(0, V // rows_per_it, zrow, 0)

    return k(qvalb_flat, qidx_flat, didx_flat, dval_pad)


def kernel(q_index_float_32, q_index, q_sparse_index, d_index, d_sparse_index,
           W, Wl, bl):
    B, L = q_index.shape
    V = W.shape[0]
    assert B % (_NS * _TILE) == 0 and V % 8 == 0

    dval = _dval_table(W, Wl, bl).reshape(V)
    Vp = ((V + _LANES - 1) // _LANES) * _LANES
    dval_pad = jnp.pad(dval, (0, Vp - V))

    qval = lax.bitcast_convert_type(q_index_float_32, jnp.int32).reshape(-1)
    qidx = q_index.astype(jnp.int32).reshape(-1)
    didx = d_index.astype(jnp.int32).reshape(-1)

    rel, d_bow = _sc_bow(qval, qidx, didx, dval_pad, V, B, L)
    return rel, d_bow


# speedup vs baseline: 1.3057x; 1.3057x over previous
import functools

import jax
import jax.numpy as jnp
from jax import lax
from jax.experimental import pallas as pl
from jax.experimental.pallas import tpu as pltpu
from jax.experimental.pallas import tpu_sc as plsc

_NC, _NS = 2, 16
_NW = _NC * _NS
_LANES = 16


def _dval_table(W, Wl, bl):
    def body(w_ref, wl_ref, bl_ref, out_ref):
        u = jnp.dot(w_ref[...], wl_ref[...], preferred_element_type=jnp.float32)
        u = jnp.maximum(u + bl_ref[...], 0.0)
        row = lax.broadcasted_iota(jnp.int32, u.shape, 0)
        out_ref[...] = jnp.where(row == 0, 0.0, u)

    V = W.shape[0]
    return pl.pallas_call(
        body,
        out_shape=jax.ShapeDtypeStruct((V, 1), jnp.float32),
    )(W, Wl, bl)


def _sc_bow(qvalb_flat, qidx_flat, didx_flat, dval_pad, V, B, L):
    CPW = B // _NW
    NG = CPW // _LANES
    Vp = dval_pad.shape[0]
    GW = _LANES * L
    UNR = 5
    assert L % UNR == 0

    mesh = plsc.VectorSubcoreMesh(
        core_axis_name="c", subcore_axis_name="s",
        num_cores=_NC, num_subcores=_NS)

    @functools.partial(
        pl.kernel,
        out_type=(jax.ShapeDtypeStruct((B,), jnp.float32),
                  jax.ShapeDtypeStruct((V, B), jnp.float32)),
        mesh=mesh,
        compiler_params=pltpu.CompilerParams(
            needs_layout_passes=False, skip_device_barrier=True),
        scratch_types=[
            pltpu.VMEM((GW,), jnp.int32),
            pltpu.VMEM((GW,), jnp.int32),
            pltpu.VMEM((Vp,), jnp.float32),
            pltpu.VMEM((V, CPW), jnp.float32),
            pltpu.VMEM((CPW,), jnp.float32),
            pltpu.SemaphoreType.DMA,
            pltpu.SemaphoreType.DMA,
            pltpu.SemaphoreType.DMA,
            pltpu.SemaphoreType.DMA,
        ])
    def k(qvalb_hbm, qidx_hbm, didx_hbm, dval_hbm, rel_hbm, dbow_hbm,
          buf_a, buf_b, dval_v, acc_v, rel_v, sem_a, sem_b, sem_d, sem_o):
        wid = lax.axis_index("s") * _NC + lax.axis_index("c")
        bs = wid * CPW
        lane = lax.iota(jnp.int32, _LANES)
        zero16 = jnp.zeros((_LANES,), jnp.float32)
        fbase = lane * L
        bufs, sems = [buf_a, buf_b], [sem_a, sem_b]

        cp_d = pltpu.async_copy(dval_hbm, dval_v, sem_d)
        cps = [pltpu.async_copy(didx_hbm.at[pl.ds(bs * L, GW)], buf_a, sem_a),
               None]

        rows_per_it = 4
        def zrow(r, carry):
            for rr in range(rows_per_it):
                for c in range(CPW // _LANES):
                    acc_v[r * rows_per_it + rr, pl.ds(c * _LANES, _LANES)] = zero16
            return carry
        lax.fori_loop(0, V // rows_per_it, zrow, 0)
        cp_d.wait()

        for g in range(NG):
            cps[g % 2].wait()
            if g + 1 < NG:
                cps[(g + 1) % 2] = pltpu.async_copy(
                    didx_hbm.at[pl.ds((bs + (g + 1) * _LANES) * L, GW)],
                    bufs[(g + 1) % 2], sems[(g + 1) % 2])
            cur = bufs[g % 2]
            jloc = g * _LANES + lane

            def dbody(i, carry):
                for u in range(UNR):
                    dtok = plsc.load_gather(cur, [fbase + (i * UNR + u)])
                    dv = plsc.load_gather(dval_v, [dtok])
                    plsc.addupdate_scatter(acc_v, [dtok, jloc], dv)
                return carry
            lax.fori_loop(0, L // UNR, dbody, 0)

        cp_out = pltpu.async_copy(acc_v, dbow_hbm.at[:, pl.ds(bs, CPW)], sem_o)

        for g in range(NG):
            goff = (bs + g * _LANES) * L
            pltpu.sync_copy(qidx_hbm.at[pl.ds(goff, GW)], buf_a)
            pltpu.sync_copy(qvalb_hbm.at[pl.ds(goff, GW)], buf_b)
            jloc = g * _LANES + lane

            def qbody(i, racc):
                for u in range(UNR):
                    qtok = plsc.load_gather(buf_a, [fbase + (i * UNR + u)])
                    qvb = plsc.load_gather(buf_b, [fbase + (i * UNR + u)])
                    dcol = plsc.load_gather(acc_v, [qtok, jloc])
                    racc = racc + plsc.bitcast(qvb, jnp.float32) * dcol
                return racc
            racc = lax.fori_loop(0, L // UNR, qbody, zero16)
            rel_v[pl.ds(g * _LANES, _LANES)] = racc

        pltpu.sync_copy(rel_v, rel_hbm.at[pl.ds(bs, CPW)])
        cp_out.wait()

    return k(qvalb_flat, qidx_flat, didx_flat, dval_pad)


def kernel(q_index_float_32, q_index, q_sparse_index, d_index, d_sparse_index,
           W, Wl, bl):
    B, L = q_index.shape
    V = W.shape[0]
    assert B % (_NW * _LANES) == 0 and V % 8 == 0

    dval = _dval_table(W, Wl, bl).reshape(V)
    Vp = ((V + _LANES - 1) // _LANES) * _LANES
    dval_pad = jnp.pad(dval, (0, Vp - V))

    qval = lax.bitcast_convert_type(q_index_float_32, jnp.int32).reshape(-1)
    qidx = q_index.astype(jnp.int32).reshape(-1)
    didx = d_index.astype(jnp.int32).reshape(-1)

    rel, d_bow = _sc_bow(qval, qidx, didx, dval_pad, V, B, L)
    return rel, d_bow
